# baseline (speedup 1.0000x reference)
import functools

import jax
import jax.numpy as jnp
from jax.experimental import pallas as pl
from jax.experimental.pallas import tpu as pltpu

_TOPK = 32
_R1 = 256


def _phase1_body(x_ref, w_ref, b_ref, bd_ref, s_ref, em_ref, d_ref, bbuf,
                 *, n_zero_tiles, n_data_tiles):
    i = pl.program_id(0)

    @pl.when(i < n_zero_tiles)
    def _():
        em_ref[...] = jnp.zeros_like(em_ref)
        d_ref[...] = jnp.broadcast_to(bd_ref[...], d_ref.shape)

    @pl.when(i < n_data_tiles)
    def _():
        enc = jnp.maximum(
            jnp.dot(x_ref[...], w_ref[...], preferred_element_type=jnp.float32)
            + b_ref[...],
            0.0,
        )
        bbuf[i % 2] = jax.lax.bitcast_convert_type(enc, jnp.int32)

    @pl.when(i > 0)
    def _():
        _phase1_bisect(bbuf[(i + 1) % 2], s_ref, i)


def _phase1_bisect(bits, s_ref, i):
    rows = bits.shape[0]

    def packed_count(ind16):
        part = ind16
        while part.shape[1] > 128:
            h = part.shape[1] // 2
            part = part[:, :h] + part[:, h:]
        return jnp.sum(part.astype(jnp.float32), axis=1, keepdims=True)

    t16 = (bits >> 16).astype(jnp.int16)

    def coarse_step(_, carry):
        lo, hi, cnt_hi = carry
        mid = lo + ((hi - lo) >> 1)
        cnt = packed_count((t16 >= mid.astype(jnp.int16)).astype(jnp.int16))
        ok = cnt >= _TOPK
        return (jnp.where(ok, mid, lo), jnp.where(ok, hi, mid),
                jnp.where(ok, cnt_hi, cnt))

    p, _, cnt_gt = jax.lax.fori_loop(
        0, 15, coarse_step,
        (jnp.zeros((rows, 1), jnp.int32),
         jnp.full((rows, 1), 1 << 15, jnp.int32),
         jnp.zeros((rows, 1), jnp.float32)),
    )
    p16 = p.astype(jnp.int16)

    need = _TOPK - cnt_gt
    low16 = (bits ^ 0x8000).astype(jnp.int16)
    ml = jnp.where(t16 == p16, low16, jnp.int16(-32768))

    def fine_step(_, carry):
        lo, hi = carry
        mid = lo + ((hi - lo) >> 1)
        cnt = packed_count((ml >= mid.astype(jnp.int16)).astype(jnp.int16))
        ok = cnt >= need
        return jnp.where(ok, mid, lo), jnp.where(ok, hi, mid)

    lf, _ = jax.lax.fori_loop(
        0, 16, fine_step,
        (jnp.full((rows, 1), -32768, jnp.int32),
         jnp.full((rows, 1), 1 << 15, jnp.int32)),
    )
    thr = (p << 16) | (lf + 32768)
    s_part = jnp.max(bits - thr, axis=0, keepdims=True)

    @pl.when(i == 1)
    def _():
        s_ref[...] = s_part

    @pl.when(i != 1)
    def _():
        s_ref[...] = jnp.maximum(s_ref[...], s_part)


def _phase2_body(x_ref, w_ref, b_ref, wd_ref, bd_ref, s_ref, _em_in, _d_in,
                 m_ref, d_ref):
    enc = jnp.maximum(
        jnp.dot(x_ref[...], w_ref[...], preferred_element_type=jnp.float32)
        + b_ref[...],
        0.0,
    )
    gate = s_ref[:, 0:1]
    masked = enc * gate
    m_ref[...] = masked
    d_ref[...] = (
        jnp.dot(masked, wd_ref[...], preferred_element_type=jnp.float32)
        + bd_ref[...]
    )


def kernel(x, W_enc, b_enc, W_dec, b_dec):
    B, F = x.shape
    H = W_enc.shape[0]
    O = W_dec.shape[0]
    W_encT = W_enc.T
    W_decT = W_dec.T
    b_enc2 = b_enc.reshape(1, H)
    b_dec2 = b_dec.reshape(1, O)

    n1 = B // _R1
    n_live = H // _R1
    n_zero = n1 - n_live
    zmap = lambda i: (n_live + jnp.minimum(i, n_zero - 1), 0)
    body1 = functools.partial(_phase1_body, n_zero_tiles=n_zero,
                              n_data_tiles=n1)
    s, em_init, dec_init = pl.pallas_call(
        body1,
        grid=(n1 + 1,),
        in_specs=[
            pl.BlockSpec((_R1, F), lambda i: (jnp.minimum(i, n1 - 1), 0)),
            pl.BlockSpec((F, H), lambda i: (0, 0)),
            pl.BlockSpec((1, H), lambda i: (0, 0)),
            pl.BlockSpec((1, O), lambda i: (0, 0)),
        ],
        out_specs=[
            pl.BlockSpec((1, H), lambda i: (0, 0)),
            pl.BlockSpec((_R1, H), zmap),
            pl.BlockSpec((_R1, O), zmap),
        ],
        out_shape=[
            jax.ShapeDtypeStruct((1, H), jnp.int32),
            jax.ShapeDtypeStruct((B, H), jnp.float32),
            jax.ShapeDtypeStruct((B, O), jnp.float32),
        ],
        scratch_shapes=[pltpu.VMEM((2, _R1, H), jnp.int32)],
    )(x, W_encT, b_enc2, b_dec2)
    s = (s >= 0).astype(jnp.float32)

    s_col = jnp.broadcast_to(s.reshape(H, 1), (H, 128))

    enc_masked, decoded = pl.pallas_call(
        _phase2_body,
        grid=(n_live,),
        in_specs=[
            pl.BlockSpec((_R1, F), lambda i: (i, 0)),
            pl.BlockSpec((F, H), lambda i: (0, 0)),
            pl.BlockSpec((1, H), lambda i: (0, 0)),
            pl.BlockSpec((H, O), lambda i: (0, 0)),
            pl.BlockSpec((1, O), lambda i: (0, 0)),
            pl.BlockSpec((_R1, 128), lambda i: (i, 0)),
            pl.BlockSpec(memory_space=pl.ANY),
            pl.BlockSpec(memory_space=pl.ANY),
        ],
        out_specs=[
            pl.BlockSpec((_R1, H), lambda i: (i, 0)),
            pl.BlockSpec((_R1, O), lambda i: (i, 0)),
        ],
        out_shape=[
            jax.ShapeDtypeStruct((B, H), jnp.float32),
            jax.ShapeDtypeStruct((B, O), jnp.float32),
        ],
        input_output_aliases={6: 0, 7: 1},
    )(x, W_encT, b_enc2, W_decT, b_dec2, s_col, em_init, dec_init)

    return enc_masked, decoded


# speedup vs baseline: 1.0456x; 1.0456x over previous
import functools

import jax
import jax.numpy as jnp
from jax.experimental import pallas as pl
from jax.experimental.pallas import tpu as pltpu

_TOPK = 32
_R1 = 256


def _phase1_body(x_ref, w_ref, b_ref, bd_ref, s_ref, em_ref, d_ref,
                 *, n_zero_tiles):
    i = pl.program_id(0)

    @pl.when(i < n_zero_tiles)
    def _():
        em_ref[...] = jnp.zeros_like(em_ref)
        d_ref[...] = jnp.broadcast_to(bd_ref[...], d_ref.shape)
    enc = jnp.maximum(
        jnp.dot(x_ref[...], w_ref[...], preferred_element_type=jnp.float32)
        + b_ref[...],
        0.0,
    )
    bits = jax.lax.bitcast_convert_type(enc, jnp.int32)
    rows = bits.shape[0]

    def packed_count(ind16):
        part = ind16
        while part.shape[1] > 128:
            h = part.shape[1] // 2
            part = part[:, :h] + part[:, h:]
        return jnp.sum(part.astype(jnp.float32), axis=1, keepdims=True)

    t16 = (bits >> 16).astype(jnp.int16)

    def coarse_step(_, carry):
        lo, hi, cnt_hi = carry
        mid = lo + ((hi - lo) >> 1)
        cnt = packed_count((t16 >= mid.astype(jnp.int16)).astype(jnp.int16))
        ok = cnt >= _TOPK
        return (jnp.where(ok, mid, lo), jnp.where(ok, hi, mid),
                jnp.where(ok, cnt_hi, cnt))

    p, _, cnt_gt = jax.lax.fori_loop(
        0, 15, coarse_step,
        (jnp.zeros((rows, 1), jnp.int32),
         jnp.full((rows, 1), 1 << 15, jnp.int32),
         jnp.zeros((rows, 1), jnp.float32)),
    )
    p16 = p.astype(jnp.int16)

    need = _TOPK - cnt_gt
    low16 = (bits ^ 0x8000).astype(jnp.int16)
    ml = jnp.where(t16 == p16, low16, jnp.int16(-32768))

    def fine_step(_, carry):
        lo, hi = carry
        mid = lo + ((hi - lo) >> 1)
        cnt = packed_count((ml >= mid.astype(jnp.int16)).astype(jnp.int16))
        ok = cnt >= need
        return jnp.where(ok, mid, lo), jnp.where(ok, hi, mid)

    lf, _ = jax.lax.fori_loop(
        0, 16, fine_step,
        (jnp.full((rows, 1), -32768, jnp.int32),
         jnp.full((rows, 1), 1 << 15, jnp.int32)),
    )
    thr = (p << 16) | (lf + 32768)
    s_part = jnp.max(bits - thr, axis=0, keepdims=True)

    @pl.when(pl.program_id(0) == 0)
    def _():
        s_ref[...] = s_part

    @pl.when(pl.program_id(0) != 0)
    def _():
        s_ref[...] = jnp.maximum(s_ref[...], s_part)


def _phase2_body(x_ref, w_ref, b_ref, wd_ref, bd_ref, s_ref, _em_in, _d_in,
                 m_ref, d_ref):
    enc = jnp.maximum(
        jnp.dot(x_ref[...], w_ref[...], preferred_element_type=jnp.float32)
        + b_ref[...],
        0.0,
    )
    gate = s_ref[:, 0:1]
    masked = enc * gate
    m_ref[...] = masked
    d_ref[...] = (
        jnp.dot(masked, wd_ref[...], preferred_element_type=jnp.float32)
        + bd_ref[...]
    )


def kernel(x, W_enc, b_enc, W_dec, b_dec):
    B, F = x.shape
    H = W_enc.shape[0]
    O = W_dec.shape[0]
    W_encT = W_enc.T
    W_decT = W_dec.T
    b_enc2 = b_enc.reshape(1, H)
    b_dec2 = b_dec.reshape(1, O)

    n1 = B // _R1
    n_live = H // _R1
    n_zero = n1 - n_live
    zmap = lambda i: (n_live + jnp.minimum(i, n_zero - 1), 0)
    body1 = functools.partial(_phase1_body, n_zero_tiles=n_zero)
    s, em_init, dec_init = pl.pallas_call(
        body1,
        grid=(n1,),
        in_specs=[
            pl.BlockSpec((_R1, F), lambda i: (i, 0)),
            pl.BlockSpec((F, H), lambda i: (0, 0)),
            pl.BlockSpec((1, H), lambda i: (0, 0)),
            pl.BlockSpec((1, O), lambda i: (0, 0)),
        ],
        out_specs=[
            pl.BlockSpec((1, H), lambda i: (0, 0)),
            pl.BlockSpec((_R1, H), zmap),
            pl.BlockSpec((_R1, O), zmap),
        ],
        out_shape=[
            jax.ShapeDtypeStruct((1, H), jnp.int32),
            jax.ShapeDtypeStruct((B, H), jnp.float32),
            jax.ShapeDtypeStruct((B, O), jnp.float32),
        ],
    )(x, W_encT, b_enc2, b_dec2)
    s = (s >= 0).astype(jnp.float32)

    s_col = jnp.broadcast_to(s.reshape(H, 1), (H, 128))

    enc_masked, decoded = pl.pallas_call(
        _phase2_body,
        grid=(n_live,),
        in_specs=[
            pl.BlockSpec((_R1, F), lambda i: (i, 0)),
            pl.BlockSpec((F, H), lambda i: (0, 0)),
            pl.BlockSpec((1, H), lambda i: (0, 0)),
            pl.BlockSpec((H, O), lambda i: (0, 0)),
            pl.BlockSpec((1, O), lambda i: (0, 0)),
            pl.BlockSpec((_R1, 128), lambda i: (i, 0)),
            pl.BlockSpec(memory_space=pl.ANY),
            pl.BlockSpec(memory_space=pl.ANY),
        ],
        out_specs=[
            pl.BlockSpec((_R1, H), lambda i: (i, 0)),
            pl.BlockSpec((_R1, O), lambda i: (i, 0)),
        ],
        out_shape=[
            jax.ShapeDtypeStruct((B, H), jnp.float32),
            jax.ShapeDtypeStruct((B, O), jnp.float32),
        ],
        input_output_aliases={6: 0, 7: 1},
    )(x, W_encT, b_enc2, W_decT, b_dec2, s_col, em_init, dec_init)

    return enc_masked, decoded


# speedup vs baseline: 1.0907x; 1.0431x over previous
import functools

import jax
import jax.numpy as jnp
from jax.experimental import pallas as pl
from jax.experimental.pallas import tpu as pltpu

_TOPK = 32
_R1 = 512
_R2 = 256


def _phase1_body(x_ref, w_ref, b_ref, bd_ref, s_ref, em_ref, d_ref,
                 *, n_zero_tiles):
    i = pl.program_id(0)

    @pl.when(i < n_zero_tiles)
    def _():
        em_ref[...] = jnp.zeros_like(em_ref)
        d_ref[...] = jnp.broadcast_to(bd_ref[...], d_ref.shape)
    enc = jnp.maximum(
        jnp.dot(x_ref[...], w_ref[...], preferred_element_type=jnp.float32)
        + b_ref[...],
        0.0,
    )
    bits = jax.lax.bitcast_convert_type(enc, jnp.int32)
    rows = bits.shape[0]

    def packed_count(ind16):
        part = ind16
        while part.shape[1] > 128:
            h = part.shape[1] // 2
            part = part[:, :h] + part[:, h:]
        return jnp.sum(part.astype(jnp.float32), axis=1, keepdims=True)

    t16 = (bits >> 16).astype(jnp.int16)

    def coarse_step(_, carry):
        lo, hi, cnt_hi = carry
        mid = lo + ((hi - lo) >> 1)
        cnt = packed_count((t16 >= mid.astype(jnp.int16)).astype(jnp.int16))
        ok = cnt >= _TOPK
        return (jnp.where(ok, mid, lo), jnp.where(ok, hi, mid),
                jnp.where(ok, cnt_hi, cnt))

    p, _, cnt_gt = jax.lax.fori_loop(
        0, 15, coarse_step,
        (jnp.zeros((rows, 1), jnp.int32),
         jnp.full((rows, 1), 1 << 15, jnp.int32),
         jnp.zeros((rows, 1), jnp.float32)),
    )
    p16 = p.astype(jnp.int16)

    need = _TOPK - cnt_gt
    low16 = (bits ^ 0x8000).astype(jnp.int16)
    ml = jnp.where(t16 == p16, low16, jnp.int16(-32768))

    def fine_step(_, carry):
        lo, hi = carry
        mid = lo + ((hi - lo) >> 1)
        cnt = packed_count((ml >= mid.astype(jnp.int16)).astype(jnp.int16))
        ok = cnt >= need
        return jnp.where(ok, mid, lo), jnp.where(ok, hi, mid)

    lf, _ = jax.lax.fori_loop(
        0, 16, fine_step,
        (jnp.full((rows, 1), -32768, jnp.int32),
         jnp.full((rows, 1), 1 << 15, jnp.int32)),
    )
    thr = (p << 16) | (lf + 32768)
    s_part = jnp.max(bits - thr, axis=0, keepdims=True)

    @pl.when(pl.program_id(0) == 0)
    def _():
        s_ref[...] = s_part

    @pl.when(pl.program_id(0) != 0)
    def _():
        s_ref[...] = jnp.maximum(s_ref[...], s_part)


def _phase2_body(x_ref, w_ref, b_ref, wd_ref, bd_ref, s_ref, _em_in, _d_in,
                 m_ref, d_ref):
    enc = jnp.maximum(
        jnp.dot(x_ref[...], w_ref[...], preferred_element_type=jnp.float32)
        + b_ref[...],
        0.0,
    )
    gate = s_ref[:, 0:1]
    masked = enc * gate
    m_ref[...] = masked
    d_ref[...] = (
        jnp.dot(masked, wd_ref[...], preferred_element_type=jnp.float32)
        + bd_ref[...]
    )


def kernel(x, W_enc, b_enc, W_dec, b_dec):
    B, F = x.shape
    H = W_enc.shape[0]
    O = W_dec.shape[0]
    W_encT = W_enc.T
    W_decT = W_dec.T
    b_enc2 = b_enc.reshape(1, H)
    b_dec2 = b_dec.reshape(1, O)

    n1 = B // _R1
    n_live = H // _R1
    _RZ = 256
    nz_live = H // _RZ
    n_zero = B // _RZ - nz_live
    zmap = lambda i: (nz_live + jnp.minimum(i, n_zero - 1), 0)
    body1 = functools.partial(_phase1_body, n_zero_tiles=n_zero)
    s, em_init, dec_init = pl.pallas_call(
        body1,
        grid=(n1,),
        in_specs=[
            pl.BlockSpec((_R1, F), lambda i: (i, 0)),
            pl.BlockSpec((F, H), lambda i: (0, 0)),
            pl.BlockSpec((1, H), lambda i: (0, 0)),
            pl.BlockSpec((1, O), lambda i: (0, 0)),
        ],
        out_specs=[
            pl.BlockSpec((1, H), lambda i: (0, 0)),
            pl.BlockSpec((_RZ, H), zmap),
            pl.BlockSpec((_RZ, O), zmap),
        ],
        out_shape=[
            jax.ShapeDtypeStruct((1, H), jnp.int32),
            jax.ShapeDtypeStruct((B, H), jnp.float32),
            jax.ShapeDtypeStruct((B, O), jnp.float32),
        ],
    )(x, W_encT, b_enc2, b_dec2)
    s = (s >= 0).astype(jnp.float32)

    s_col = jnp.broadcast_to(s.reshape(H, 1), (H, 128))

    enc_masked, decoded = pl.pallas_call(
        _phase2_body,
        grid=(H // _R2,),
        in_specs=[
            pl.BlockSpec((_R2, F), lambda i: (i, 0)),
            pl.BlockSpec((F, H), lambda i: (0, 0)),
            pl.BlockSpec((1, H), lambda i: (0, 0)),
            pl.BlockSpec((H, O), lambda i: (0, 0)),
            pl.BlockSpec((1, O), lambda i: (0, 0)),
            pl.BlockSpec((_R2, 128), lambda i: (i, 0)),
            pl.BlockSpec(memory_space=pl.ANY),
            pl.BlockSpec(memory_space=pl.ANY),
        ],
        out_specs=[
            pl.BlockSpec((_R2, H), lambda i: (i, 0)),
            pl.BlockSpec((_R2, O), lambda i: (i, 0)),
        ],
        out_shape=[
            jax.ShapeDtypeStruct((B, H), jnp.float32),
            jax.ShapeDtypeStruct((B, O), jnp.float32),
        ],
        input_output_aliases={6: 0, 7: 1},
    )(x, W_encT, b_enc2, W_decT, b_dec2, s_col, em_init, dec_init)

    return enc_masked, decoded
